# Initial kernel scaffold; baseline (speedup 1.0000x reference)
import jax
import jax.numpy as jnp
from jax.experimental import pallas as pl


def kernel(x, W, b):
    raise NotImplementedError("write your pallas kernel here")


# speedup vs baseline: 5.7591x; 5.7591x over previous
import jax
import jax.numpy as jnp
from jax.experimental import pallas as pl
from jax.experimental.pallas import tpu as pltpu

_K = 64
_ROWS = 16
_ITERS = 34


def _block(x_ref, wt_ref, b_ref, o_ref, keys_ref):
    acts = jnp.dot(x_ref[...], wt_ref[...],
                   preferred_element_type=jnp.float32) + b_ref[...]
    acts = jnp.where(acts >= 0, acts, 0.1 * acts)

    bits = jax.lax.bitcast_convert_type(acts, jnp.int32)
    keys = jnp.where(bits < 0, bits ^ jnp.int32(0x7FFFFFFF), bits)
    keys_ref[...] = keys

    lo = jnp.min(keys, axis=1, keepdims=True)
    hi = jnp.max(keys, axis=1, keepdims=True)

    def body(_, carry):
        lo, hi = carry
        mid = (lo >> 1) + (hi >> 1) + ((lo | hi) & 1)
        cnt = jnp.sum((keys_ref[...] >= mid).astype(jnp.int32),
                      axis=1, keepdims=True)
        ge = cnt >= _K
        return jnp.where(ge, mid, lo), jnp.where(ge, hi, mid - 1)

    lo, hi = jax.lax.fori_loop(0, _ITERS, body, (lo, hi))

    o_ref[...] = jnp.where(keys_ref[...] >= lo, acts, 0.0)


def kernel(x, W, b):
    B, IN = x.shape
    OUT = W.shape[0]
    Wt = W.T
    b2 = b.reshape(1, OUT)
    grid = (B // _ROWS,)
    return pl.pallas_call(
        _block,
        grid=grid,
        in_specs=[
            pl.BlockSpec((_ROWS, IN), lambda i: (i, 0)),
            pl.BlockSpec((IN, OUT), lambda i: (0, 0)),
            pl.BlockSpec((1, OUT), lambda i: (0, 0)),
        ],
        out_specs=pl.BlockSpec((_ROWS, OUT), lambda i: (i, 0)),
        out_shape=jax.ShapeDtypeStruct((B, OUT), jnp.float32),
        scratch_shapes=[pltpu.VMEM((_ROWS, OUT), jnp.int32)],
        compiler_params=pltpu.CompilerParams(
            vmem_limit_bytes=120 * 1024 * 1024),
    )(x, Wt, b2)


# speedup vs baseline: 5.7937x; 1.0060x over previous
import jax
import jax.numpy as jnp
from jax.experimental import pallas as pl
from jax.experimental.pallas import tpu as pltpu

_K = 64
_ROWS = 16
_COLS = 4096
_ITERS = 33


def _dense(x_ref, wt_ref, b_ref, a_ref):
    acts = jnp.dot(x_ref[...], wt_ref[...],
                   preferred_element_type=jnp.float32) + b_ref[...]
    a_ref[...] = jnp.where(acts >= 0, acts, 0.1 * acts)


def _select(a_ref, o_ref, keys_ref):
    acts = a_ref[...]
    bits = jax.lax.bitcast_convert_type(acts, jnp.int32)
    keys = jnp.where(bits < 0, bits ^ jnp.int32(0x7FFFFFFF), bits)
    keys_ref[...] = keys

    lo = jnp.min(keys, axis=1, keepdims=True)
    hi = jnp.max(keys, axis=1, keepdims=True)

    def body(_, carry):
        lo, hi = carry
        mid = (lo >> 1) + (hi >> 1) + ((lo | hi) & 1)
        cnt = jnp.sum((keys_ref[...] >= mid).astype(jnp.int32),
                      axis=1, keepdims=True)
        ge = cnt >= _K
        return jnp.where(ge, mid, lo), jnp.where(ge, hi, mid - 1)

    lo, hi = jax.lax.fori_loop(0, _ITERS, body, (lo, hi))

    o_ref[...] = jnp.where(keys_ref[...] >= lo, acts, 0.0)


def kernel(x, W, b):
    B, IN = x.shape
    OUT = W.shape[0]
    Wt = W.T
    b2 = b.reshape(1, OUT)

    acts = pl.pallas_call(
        _dense,
        grid=(pl.cdiv(OUT, _COLS),),
        in_specs=[
            pl.BlockSpec((B, IN), lambda j: (0, 0)),
            pl.BlockSpec((IN, _COLS), lambda j: (0, j)),
            pl.BlockSpec((1, _COLS), lambda j: (0, j)),
        ],
        out_specs=pl.BlockSpec((B, _COLS), lambda j: (0, j)),
        out_shape=jax.ShapeDtypeStruct((B, OUT), jnp.float32),
        compiler_params=pltpu.CompilerParams(
            vmem_limit_bytes=120 * 1024 * 1024),
    )(x, Wt, b2)

    return pl.pallas_call(
        _select,
        grid=(B // _ROWS,),
        in_specs=[pl.BlockSpec((_ROWS, OUT), lambda i: (i, 0))],
        out_specs=pl.BlockSpec((_ROWS, OUT), lambda i: (i, 0)),
        out_shape=jax.ShapeDtypeStruct((B, OUT), jnp.float32),
        scratch_shapes=[pltpu.VMEM((_ROWS, OUT), jnp.int32)],
        compiler_params=pltpu.CompilerParams(
            vmem_limit_bytes=120 * 1024 * 1024),
    )(acts)
